# Initial kernel scaffold; baseline (speedup 1.0000x reference)
import jax
import jax.numpy as jnp
from jax.experimental import pallas as pl


def kernel(x, edge_index, pos, s_emb_W, s_emb_b, frac_alpha, frac_gate_W, frac_gate_b, convs):
    raise NotImplementedError("write your pallas kernel here")


# speedup vs baseline: 12.4243x; 12.4243x over previous
import functools

import jax
import jax.numpy as jnp
from jax import lax
from jax.experimental import pallas as pl
from jax.experimental.pallas import tpu as pltpu
from jax.experimental.pallas import tpu_sc as plsc

N = 10000
E = 320000
H = 128
WIN = 5
K = E // N

NC = 2
NS = 16
NW = NC * NS
EPW = E // NW
CH = 80
NCHUNK = EPW // CH

_MESH = plsc.VectorSubcoreMesh(core_axis_name="c", subcore_axis_name="s")
_SC_PARAMS = pltpu.CompilerParams(needs_layout_passes=False)


def _wid():
    c = lax.axis_index("c")
    s = lax.axis_index("s")
    return s * NC + c, c, s


_BLK = 400
_GRID = N // _BLK


def _embed_body(x_ref, wt_ref, b_ref, gw_ref, gb_ref, wlt_ref, bl_ref,
                wrt_ref, br_ref, s_ref, gate_ref, xl_ref, xr_ref):
    s = jnp.dot(x_ref[...], wt_ref[...], preferred_element_type=jnp.float32)
    s = jnp.clip(s + b_ref[...], -10.0, 10.0)
    s_ref[...] = s
    gl = jnp.sum(s * gw_ref[...], axis=1, keepdims=True) + gb_ref[...]
    gate_ref[...] = 1.0 / (1.0 + jnp.exp(-gl))
    xl_ref[...] = jnp.dot(s, wlt_ref[...], preferred_element_type=jnp.float32) + bl_ref[...]
    xr_ref[...] = jnp.dot(s, wrt_ref[...], preferred_element_type=jnp.float32) + br_ref[...]


def _tc_embed(x, wt, b, gw, gb, wlt, bl, wrt, br):
    row = pl.BlockSpec((_BLK, H), lambda i: (i, 0))
    w128 = pl.BlockSpec((H, H), lambda i: (0, 0))
    v128 = pl.BlockSpec((1, H), lambda i: (0, 0))
    v1 = pl.BlockSpec((1, 1), lambda i: (0, 0))
    col = pl.BlockSpec((_BLK, 1), lambda i: (i, 0))
    return pl.pallas_call(
        _embed_body,
        grid=(_GRID,),
        in_specs=[row, w128, v128, v128, v1, w128, v128, w128, v128],
        out_specs=[row, col, row, row],
        out_shape=[
            jax.ShapeDtypeStruct((N, H), jnp.float32),
            jax.ShapeDtypeStruct((N, 1), jnp.float32),
            jax.ShapeDtypeStruct((N, H), jnp.float32),
            jax.ShapeDtypeStruct((N, H), jnp.float32),
        ],
    )(x, wt, b, gw, gb, wlt, bl, wrt, br)


def _mid_body(s_ref, o0_ref, o1_ref, pb_ref, wlt_ref, bl_ref, wrt_ref, br_ref,
              s_out_ref, xl_ref, xr_ref):
    s = s_ref[...] + o0_ref[...] + o1_ref[...] + pb_ref[...]
    s = jnp.maximum(jnp.clip(s, -20.0, 20.0), 0.0)
    s_out_ref[...] = s
    xl_ref[...] = jnp.dot(s, wlt_ref[...], preferred_element_type=jnp.float32) + bl_ref[...]
    xr_ref[...] = jnp.dot(s, wrt_ref[...], preferred_element_type=jnp.float32) + br_ref[...]


def _tc_mid(s, o0, o1, pb, wlt, bl, wrt, br):
    row = pl.BlockSpec((_BLK, H), lambda i: (i, 0))
    w128 = pl.BlockSpec((H, H), lambda i: (0, 0))
    v128 = pl.BlockSpec((1, H), lambda i: (0, 0))
    return pl.pallas_call(
        _mid_body,
        grid=(_GRID,),
        in_specs=[row, row, row, v128, w128, v128, w128, v128],
        out_specs=[row, row, row],
        out_shape=[
            jax.ShapeDtypeStruct((N, H), jnp.float32),
            jax.ShapeDtypeStruct((N, H), jnp.float32),
            jax.ShapeDtypeStruct((N, H), jnp.float32),
        ],
    )(s, o0, o1, pb, wlt, bl, wrt, br)


def _post_body(s_ref, o0_ref, o1_ref, pb_ref, s_out_ref):
    s = s_ref[...] + o0_ref[...] + o1_ref[...] + pb_ref[...]
    s_out_ref[...] = jnp.maximum(jnp.clip(s, -20.0, 20.0), 0.0)


def _tc_post(s, o0, o1, pb):
    row = pl.BlockSpec((_BLK, H), lambda i: (i, 0))
    v128 = pl.BlockSpec((1, H), lambda i: (0, 0))
    return pl.pallas_call(
        _post_body,
        grid=(_GRID,),
        in_specs=[row, row, row, v128],
        out_specs=row,
        out_shape=jax.ShapeDtypeStruct((N, H), jnp.float32),
    )(s, o0, o1, pb)


def _rsqrt(x):
    i = lax.bitcast_convert_type(x, jnp.int32)
    i = jnp.int32(0x5F3759DF) - lax.shift_right_arithmetic(i, 1)
    y = lax.bitcast_convert_type(i, jnp.float32)
    for _ in range(3):
        y = y * (1.5 - 0.5 * x * y * y)
    return y


def _dist_body(src_hbm, dst_hbm, pos_hbm, gate_hbm, c_hbm, dist_hbm,
               pos_v, gate_v, c_v, sidx, didx, dbuf):
    wid, _, _ = _wid()
    pltpu.sync_copy(pos_hbm, pos_v)
    pltpu.sync_copy(gate_hbm, gate_v)
    pltpu.sync_copy(c_hbm, c_v)
    cvec = c_v[...]
    ngroups = jnp.where(wid < 16, (N // NW) + 1, N // NW)

    def group(t, carry):
        g = wid + t * NW
        off = g * K
        pltpu.sync_copy(src_hbm.at[pl.ds(off, K)], sidx)
        pltpu.sync_copy(dst_hbm.at[pl.ds(off, K)], didx)
        ds = []
        for r in range(K // 16):
            si = sidx[pl.ds(r * 16, 16)] * 3
            di = didx[pl.ds(r * 16, 16)] * 3
            dx = plsc.load_gather(pos_v, [si]) - plsc.load_gather(pos_v, [di])
            dy = plsc.load_gather(pos_v, [si + 1]) - plsc.load_gather(pos_v, [di + 1])
            dz = plsc.load_gather(pos_v, [si + 2]) - plsc.load_gather(pos_v, [di + 2])
            d2 = dx * dx + dy * dy + dz * dz + 1e-12
            ds.append(d2 * _rsqrt(d2))
        wsum = jnp.sum(cvec * ds[0])
        gv = gate_v[pl.ds(g, 16)]
        filt = gv[0] * wsum
        for r in range(K // 16):
            dbuf[pl.ds(r * 16, 16)] = ds[r] + filt
        pltpu.sync_copy(dbuf, dist_hbm.at[pl.ds(off, K)])
        return carry

    lax.fori_loop(0, ngroups, group, 0, unroll=False)


def _sc_dist(src, dst, pos_flat, gate_flat, c_pad):
    return pl.kernel(
        _dist_body,
        out_type=jax.ShapeDtypeStruct((E,), jnp.float32),
        mesh=_MESH,
        compiler_params=_SC_PARAMS,
        scratch_types=[
            pltpu.VMEM((3 * N,), jnp.float32),
            pltpu.VMEM((N + 16,), jnp.float32),
            pltpu.VMEM((16,), jnp.float32),
            pltpu.VMEM((K,), jnp.int32),
            pltpu.VMEM((K,), jnp.int32),
            pltpu.VMEM((K,), jnp.float32),
        ],
    )(src, dst, pos_flat, gate_flat, c_pad)


def _p1_body(src_hbm, dst_hbm, dist_hbm, xl_hbm, xr_hbm, wa_hbm,
             logits_hbm, tmax_hbm,
             wa_v, mx_v, sidx, didx, dist_v, lg_v, xlr, xrr,
             s_si, s_di, s_d, s_g1, s_g2, s_lg):
    wid, _, _ = _wid()
    base = wid * EPW
    pltpu.sync_copy(wa_hbm, wa_v)
    lane = lax.broadcasted_iota(jnp.int32, (16,), 0)

    def start_idx(c, k):
        off = base + c * CH
        pltpu.async_copy(src_hbm.at[pl.ds(off, CH)], sidx.at[k], s_si.at[k])
        pltpu.async_copy(dst_hbm.at[pl.ds(off, CH)], didx.at[k], s_di.at[k])

    def wait_idx(k):
        pltpu.make_async_copy(src_hbm.at[pl.ds(0, CH)], sidx.at[k], s_si.at[k]).wait()
        pltpu.make_async_copy(dst_hbm.at[pl.ds(0, CH)], didx.at[k], s_di.at[k]).wait()

    def start_dist(c, k):
        off = base + c * CH
        pltpu.async_copy(dist_hbm.at[pl.ds(off, CH)], dist_v.at[k], s_d.at[k])

    def wait_dist(k):
        pltpu.make_async_copy(dist_hbm.at[pl.ds(0, CH)], dist_v.at[k], s_d.at[k]).wait()

    def start_gather(k):
        pltpu.async_copy(xl_hbm.at[sidx.at[k]], xlr.at[k], s_g1.at[k])
        pltpu.async_copy(xr_hbm.at[didx.at[k]], xrr.at[k], s_g2.at[k])

    def wait_gather(k):
        pltpu.make_async_copy(xl_hbm.at[sidx.at[k]], xlr.at[k], s_g1.at[k]).wait()
        pltpu.make_async_copy(xr_hbm.at[didx.at[k]], xrr.at[k], s_g2.at[k]).wait()

    def wait_lg(k):
        pltpu.make_async_copy(lg_v.at[k], logits_hbm.at[pl.ds(0, CH)], s_lg.at[k]).wait()

    def compute(c, k, lmax):
        def grp(r, lmax):
            d_vec = dist_v[k, pl.ds(r * 16, 16)]
            lgvec = jnp.zeros((16,), jnp.float32)
            for l in range(16):
                e = r * 16 + l
                d = d_vec[l]
                acc = jnp.zeros((16,), jnp.float32)
                for j in range(H // 16):
                    z = xlr[k, e, pl.ds(j * 16, 16)] \
                        + xrr[k, e, pl.ds(j * 16, 16)] \
                        + wa_v[pl.ds(j * 16, 16)] * d
                    m = jnp.maximum(z, 0.2 * z)
                    acc = acc + wa_v[pl.ds(H + j * 16, 16)] * m
                lg = jnp.sum(acc)
                lgvec = jnp.where(lane == l, lg, lgvec)
                lmax = jnp.maximum(lmax, lg)
            lg_v[k, pl.ds(r * 16, 16)] = lgvec
            return lmax

        lmax = lax.fori_loop(0, CH // 16, grp, lmax, unroll=False)
        off = base + c * CH
        pltpu.async_copy(lg_v.at[k], logits_hbm.at[pl.ds(off, CH)], s_lg.at[k])
        return lmax

    def process(c, k, lmax, last=False):
        other = 1 - k
        if not last:
            wait_idx(other)
            start_gather(other)
        wait_gather(k)
        wait_dist(k)

        @pl.when(c + 2 < NCHUNK)
        def _starts():
            start_idx(c + 2, k)

        @pl.when(c >= 2)
        def _wlg():
            wait_lg(k)
        lmax = compute(c, k, lmax)

        @pl.when(c + 2 < NCHUNK)
        def _startd():
            start_dist(c + 2, k)
        return lmax

    start_idx(0, 0)
    start_idx(1, 1)
    start_dist(0, 0)
    start_dist(1, 1)
    wait_idx(0)
    start_gather(0)

    def pair(i, lmax):
        lmax = process(2 * i, 0, lmax)
        lmax = process(2 * i + 1, 1, lmax)
        return lmax

    lmax = lax.fori_loop(0, (NCHUNK - 1) // 2, pair, jnp.float32(-3e38),
                         unroll=False)
    lmax = process(NCHUNK - 1, 0, lmax, last=True)
    wait_lg(0)
    wait_lg(1)
    mx_v[...] = jnp.full((16,), lmax, jnp.float32)
    pltpu.sync_copy(mx_v, tmax_hbm.at[pl.ds(wid * 16, 16)])


def _sc_p1(src, dst, dist, xl, xr, wa):
    return pl.kernel(
        _p1_body,
        out_type=[
            jax.ShapeDtypeStruct((E,), jnp.float32),
            jax.ShapeDtypeStruct((NW * 16,), jnp.float32),
        ],
        mesh=_MESH,
        compiler_params=_SC_PARAMS,
        scratch_types=[
            pltpu.VMEM((2 * H,), jnp.float32),
            pltpu.VMEM((16,), jnp.float32),
            pltpu.VMEM((2, CH), jnp.int32),
            pltpu.VMEM((2, CH), jnp.int32),
            pltpu.VMEM((2, CH), jnp.float32),
            pltpu.VMEM((2, CH), jnp.float32),
            pltpu.VMEM((2, CH, H), jnp.float32),
            pltpu.VMEM((2, CH, H), jnp.float32),
            pltpu.SemaphoreType.DMA((2,)),
            pltpu.SemaphoreType.DMA((2,)),
            pltpu.SemaphoreType.DMA((2,)),
            pltpu.SemaphoreType.DMA((2,)),
            pltpu.SemaphoreType.DMA((2,)),
            pltpu.SemaphoreType.DMA((2,)),
        ],
    )(src, dst, dist, xl, xr, wa)


def _global_max(tm_v):
    m = tm_v[pl.ds(0, 16)]
    for i in range(1, NW):
        m = jnp.maximum(m, tm_v[pl.ds(i * 16, 16)])
    return jnp.max(m)


def _p2_body(dst_hbm, logits_hbm, tmax_hbm, esum_hbm,
             tm_v, didx, lg_v, ex_v, zb, s_di, s_lg, esp):
    wid, c, s = _wid()
    base = wid * EPW
    pltpu.sync_copy(tmax_hbm, tm_v)
    cmax = _global_max(tm_v)

    @pl.when(s == 0)
    def _zero():
        def zrow(i, carry):
            zb[pl.ds(i * 16, 16)] = jnp.zeros((16,), jnp.float32)
            return carry
        lax.fori_loop(0, N // 16, zrow, 0, unroll=False)
        pltpu.sync_copy(zb, esp)

    def start_loads(t, k):
        off = base + t * CH
        pltpu.async_copy(dst_hbm.at[pl.ds(off, CH)], didx.at[k], s_di.at[k])
        pltpu.async_copy(logits_hbm.at[pl.ds(off, CH)], lg_v.at[k], s_lg.at[k])

    def wait_loads(k):
        pltpu.make_async_copy(dst_hbm.at[pl.ds(0, CH)], didx.at[k], s_di.at[k]).wait()
        pltpu.make_async_copy(logits_hbm.at[pl.ds(0, CH)], lg_v.at[k], s_lg.at[k]).wait()

    start_loads(0, 0)
    start_loads(1, 1)
    plsc.subcore_barrier()

    def process(t, k):
        wait_loads(k)
        for r in range(CH // 16):
            ex_v[k, pl.ds(r * 16, 16)] = jnp.exp(lg_v[k, pl.ds(r * 16, 16)] - cmax)
        pltpu.sync_copy(ex_v.at[k], esp.at[didx.at[k]], add=True)

        @pl.when(t + 2 < NCHUNK)
        def _starts():
            start_loads(t + 2, k)

    def pair(i, carry):
        process(2 * i, 0)
        process(2 * i + 1, 1)
        return carry

    lax.fori_loop(0, (NCHUNK - 1) // 2, pair, 0, unroll=False)
    process(NCHUNK - 1, 0)
    plsc.subcore_barrier()

    @pl.when(s < 15)
    def _drain():
        pltpu.sync_copy(esp.at[pl.ds(s * 624, 624)], zb.at[pl.ds(0, 624)])
        pltpu.sync_copy(zb.at[pl.ds(0, 624)],
                        esum_hbm.at[pl.ds(c * N + s * 624, 624)])

    @pl.when(s == 15)
    def _drain_last():
        pltpu.sync_copy(esp.at[pl.ds(15 * 624, N - 15 * 624)],
                        zb.at[pl.ds(0, N - 15 * 624)])
        pltpu.sync_copy(zb.at[pl.ds(0, N - 15 * 624)],
                        esum_hbm.at[pl.ds(c * N + 15 * 624, N - 15 * 624)])


def _sc_p2(dst, logits, tmax):
    return pl.kernel(
        _p2_body,
        out_type=jax.ShapeDtypeStruct((NC * N,), jnp.float32),
        mesh=_MESH,
        compiler_params=_SC_PARAMS,
        scratch_types=[
            pltpu.VMEM((NW * 16,), jnp.float32),
            pltpu.VMEM((2, CH), jnp.int32),
            pltpu.VMEM((2, CH), jnp.float32),
            pltpu.VMEM((2, CH), jnp.float32),
            pltpu.VMEM((N,), jnp.float32),
            pltpu.SemaphoreType.DMA((2,)),
            pltpu.SemaphoreType.DMA((2,)),
            pltpu.VMEM_SHARED((N,), jnp.float32),
        ],
    )(dst, logits, tmax)


def _p3_body(src_hbm, dst_hbm, logits_hbm, tmax_hbm, esum_hbm, xl_hbm,
             out_hbm,
             tm_v, es_v, tmp_v, zb, sidx, didx, lg_v, xlr,
             s_si, s_di, s_lg, s_g1, osp):
    wid, c, s = _wid()
    base = wid * EPW
    pltpu.sync_copy(tmax_hbm, tm_v)
    cmax = _global_max(tm_v)

    pltpu.sync_copy(esum_hbm.at[pl.ds(0, N)], es_v)
    pltpu.sync_copy(esum_hbm.at[pl.ds(N, N)], tmp_v)

    def comb(i, carry):
        es_v[pl.ds(i * 16, 16)] = (es_v[pl.ds(i * 16, 16)]
                                   + tmp_v[pl.ds(i * 16, 16)] + 1e-16)
        return carry
    lax.fori_loop(0, N // 16, comb, 0, unroll=False)

    def zrow(i, carry):
        for j in range(H // 16):
            zb[i, pl.ds(j * 16, 16)] = jnp.zeros((16,), jnp.float32)
        return carry
    lax.fori_loop(0, 48, zrow, 0, unroll=False)
    for q in range(13):
        pltpu.sync_copy(zb, osp.at[pl.ds(s * 624 + q * 48, 48), :])

    @pl.when(s == 15)
    def _zero_tail():
        pltpu.sync_copy(zb.at[pl.ds(0, 32), :], osp.at[pl.ds(N - 32, 32), :])

    def start_loads(t, k):
        off = base + t * CH
        pltpu.async_copy(src_hbm.at[pl.ds(off, CH)], sidx.at[k], s_si.at[k])
        pltpu.async_copy(dst_hbm.at[pl.ds(off, CH)], didx.at[k], s_di.at[k])
        pltpu.async_copy(logits_hbm.at[pl.ds(off, CH)], lg_v.at[k], s_lg.at[k])

    def wait_loads(k):
        pltpu.make_async_copy(src_hbm.at[pl.ds(0, CH)], sidx.at[k], s_si.at[k]).wait()
        pltpu.make_async_copy(dst_hbm.at[pl.ds(0, CH)], didx.at[k], s_di.at[k]).wait()
        pltpu.make_async_copy(logits_hbm.at[pl.ds(0, CH)], lg_v.at[k], s_lg.at[k]).wait()

    def start_gather(k):
        pltpu.async_copy(xl_hbm.at[sidx.at[k]], xlr.at[k], s_g1.at[k])

    def wait_gather(k):
        pltpu.make_async_copy(xl_hbm.at[sidx.at[k]], xlr.at[k], s_g1.at[k]).wait()

    start_loads(0, 0)
    start_loads(1, 1)
    wait_loads(0)
    start_gather(0)
    plsc.subcore_barrier()

    def process(t, k, last=False):
        other = 1 - k
        if not last:
            wait_loads(other)
            start_gather(other)
        wait_gather(k)

        def grp(r, carry):
            ex = jnp.exp(lg_v[k, pl.ds(r * 16, 16)] - cmax)
            den = plsc.load_gather(es_v, [didx[k, pl.ds(r * 16, 16)]])
            a_vec = ex / den
            for l in range(16):
                e = r * 16 + l
                a = a_vec[l]
                for j in range(H // 16):
                    xlr[k, e, pl.ds(j * 16, 16)] = a * xlr[k, e, pl.ds(j * 16, 16)]
            return carry
        lax.fori_loop(0, CH // 16, grp, 0, unroll=False)
        pltpu.sync_copy(xlr.at[k], osp.at[didx.at[k]], add=True)

        @pl.when(t + 2 < NCHUNK)
        def _starts():
            start_loads(t + 2, k)

    def pair(i, carry):
        process(2 * i, 0)
        process(2 * i + 1, 1)
        return carry

    lax.fori_loop(0, (NCHUNK - 1) // 2, pair, 0, unroll=False)
    process(NCHUNK - 1, 0, last=True)
    plsc.subcore_barrier()

    for q in range(13):
        pltpu.sync_copy(osp.at[pl.ds(s * 624 + q * 48, 48), :], zb)
        pltpu.sync_copy(zb, out_hbm.at[pl.ds(c * N + s * 624 + q * 48, 48), :])

    @pl.when(s == 15)
    def _drain_tail():
        pltpu.sync_copy(osp.at[pl.ds(N - 32, 32), :], zb.at[pl.ds(0, 32), :])
        pltpu.sync_copy(zb.at[pl.ds(0, 32), :],
                        out_hbm.at[pl.ds(c * N + N - 32, 32), :])


def _sc_p3(src, dst, logits, tmax, esum, xl):
    return pl.kernel(
        _p3_body,
        out_type=jax.ShapeDtypeStruct((NC * N, H), jnp.float32),
        mesh=_MESH,
        compiler_params=_SC_PARAMS,
        scratch_types=[
            pltpu.VMEM((NW * 16,), jnp.float32),
            pltpu.VMEM((N,), jnp.float32),
            pltpu.VMEM((N,), jnp.float32),
            pltpu.VMEM((48, H), jnp.float32),
            pltpu.VMEM((2, CH), jnp.int32),
            pltpu.VMEM((2, CH), jnp.int32),
            pltpu.VMEM((2, CH), jnp.float32),
            pltpu.VMEM((2, CH, H), jnp.float32),
            pltpu.SemaphoreType.DMA((2,)),
            pltpu.SemaphoreType.DMA((2,)),
            pltpu.SemaphoreType.DMA((2,)),
            pltpu.SemaphoreType.DMA((2,)),
            pltpu.VMEM_SHARED((N, H), jnp.float32),
        ],
    )(src, dst, logits, tmax, esum, xl)


def kernel(x, edge_index, pos, s_emb_W, s_emb_b, frac_alpha, frac_gate_W,
           frac_gate_b, convs):
    src = edge_index[0]
    dst = edge_index[1]
    pos_flat = pos.reshape(-1)

    coeffs = [jnp.ones((), jnp.float32)]
    b = jnp.ones((), jnp.float32)
    for kk in range(1, WIN):
        b = b * (frac_alpha - (kk - 1)) / kk
        coeffs.append(((-1.0) ** kk) * b)
    c_pad = jnp.concatenate([jnp.stack(coeffs),
                             jnp.zeros((16 - WIN,), jnp.float32)])

    p0 = convs[0]
    s, gate, xl, xr = _tc_embed(
        x, s_emb_W.T, s_emb_b.reshape(1, H), frac_gate_W.reshape(1, H),
        frac_gate_b.reshape(1, 1), p0["Wl"].T, p0["bl"].reshape(1, H),
        p0["Wr"].T, p0["br"].reshape(1, H))

    gate_pad = jnp.concatenate([gate.reshape(-1), jnp.zeros((16,), jnp.float32)])
    dist = _sc_dist(src, dst, pos_flat, gate_pad, c_pad)

    outp = None
    for i, p in enumerate(convs):
        if i > 0:
            s, xl, xr = _tc_mid(s, outp[:N], outp[N:],
                                convs[i - 1]["bias"].reshape(1, H),
                                p["Wl"].T, p["bl"].reshape(1, H),
                                p["Wr"].T, p["br"].reshape(1, H))
        wa = jnp.concatenate([p["We"][:, 0], p["att"]])
        logits, tmax = _sc_p1(src, dst, dist, xl, xr, wa)
        esum = _sc_p2(dst, logits, tmax)
        outp = _sc_p3(src, dst, logits, tmax, esum, xl)

    s = _tc_post(s, outp[:N], outp[N:], convs[-1]["bias"].reshape(1, H))
    v = jnp.zeros((N, 16, 3), jnp.float32)
    return (s, v)


# speedup vs baseline: 14.6629x; 1.1802x over previous
import functools

import jax
import jax.numpy as jnp
from jax import lax
from jax.experimental import pallas as pl
from jax.experimental.pallas import tpu as pltpu
from jax.experimental.pallas import tpu_sc as plsc

N = 10000
E = 320000
H = 128
WIN = 5
K = E // N

NC = 2
NS = 16
NW = NC * NS
EPW = E // NW
CH = 80
NCHUNK = EPW // CH

_MESH = plsc.VectorSubcoreMesh(core_axis_name="c", subcore_axis_name="s")
_SC_PARAMS = pltpu.CompilerParams(needs_layout_passes=False)


def _wid():
    c = lax.axis_index("c")
    s = lax.axis_index("s")
    return s * NC + c, c, s


_BLK = 400
_GRID = N // _BLK


def _embed_body(x_ref, wt_ref, b_ref, gw_ref, gb_ref, wlt_ref, bl_ref,
                wrt_ref, br_ref, s_ref, gate_ref, xl_ref, xr_ref):
    s = jnp.dot(x_ref[...], wt_ref[...], preferred_element_type=jnp.float32)
    s = jnp.clip(s + b_ref[...], -10.0, 10.0)
    s_ref[...] = s
    gl = jnp.sum(s * gw_ref[...], axis=1, keepdims=True) + gb_ref[...]
    gate_ref[...] = 1.0 / (1.0 + jnp.exp(-gl))
    xl_ref[...] = jnp.dot(s, wlt_ref[...], preferred_element_type=jnp.float32) + bl_ref[...]
    xr_ref[...] = jnp.dot(s, wrt_ref[...], preferred_element_type=jnp.float32) + br_ref[...]


def _tc_embed(x, wt, b, gw, gb, wlt, bl, wrt, br):
    row = pl.BlockSpec((_BLK, H), lambda i: (i, 0))
    w128 = pl.BlockSpec((H, H), lambda i: (0, 0))
    v128 = pl.BlockSpec((1, H), lambda i: (0, 0))
    v1 = pl.BlockSpec((1, 1), lambda i: (0, 0))
    col = pl.BlockSpec((_BLK, 1), lambda i: (i, 0))
    return pl.pallas_call(
        _embed_body,
        grid=(_GRID,),
        in_specs=[row, w128, v128, v128, v1, w128, v128, w128, v128],
        out_specs=[row, col, row, row],
        out_shape=[
            jax.ShapeDtypeStruct((N, H), jnp.float32),
            jax.ShapeDtypeStruct((N, 1), jnp.float32),
            jax.ShapeDtypeStruct((N, H), jnp.float32),
            jax.ShapeDtypeStruct((N, H), jnp.float32),
        ],
    )(x, wt, b, gw, gb, wlt, bl, wrt, br)


def _mid_body(s_ref, o0_ref, o1_ref, pb_ref, wlt_ref, bl_ref, wrt_ref, br_ref,
              s_out_ref, xl_ref, xr_ref):
    s = s_ref[...] + o0_ref[...] + o1_ref[...] + pb_ref[...]
    s = jnp.maximum(jnp.clip(s, -20.0, 20.0), 0.0)
    s_out_ref[...] = s
    xl_ref[...] = jnp.dot(s, wlt_ref[...], preferred_element_type=jnp.float32) + bl_ref[...]
    xr_ref[...] = jnp.dot(s, wrt_ref[...], preferred_element_type=jnp.float32) + br_ref[...]


def _tc_mid(s, o0, o1, pb, wlt, bl, wrt, br):
    row = pl.BlockSpec((_BLK, H), lambda i: (i, 0))
    w128 = pl.BlockSpec((H, H), lambda i: (0, 0))
    v128 = pl.BlockSpec((1, H), lambda i: (0, 0))
    return pl.pallas_call(
        _mid_body,
        grid=(_GRID,),
        in_specs=[row, row, row, v128, w128, v128, w128, v128],
        out_specs=[row, row, row],
        out_shape=[
            jax.ShapeDtypeStruct((N, H), jnp.float32),
            jax.ShapeDtypeStruct((N, H), jnp.float32),
            jax.ShapeDtypeStruct((N, H), jnp.float32),
        ],
    )(s, o0, o1, pb, wlt, bl, wrt, br)


def _post_body(s_ref, o0_ref, o1_ref, pb_ref, s_out_ref):
    s = s_ref[...] + o0_ref[...] + o1_ref[...] + pb_ref[...]
    s_out_ref[...] = jnp.maximum(jnp.clip(s, -20.0, 20.0), 0.0)


def _tc_post(s, o0, o1, pb):
    row = pl.BlockSpec((_BLK, H), lambda i: (i, 0))
    v128 = pl.BlockSpec((1, H), lambda i: (0, 0))
    return pl.pallas_call(
        _post_body,
        grid=(_GRID,),
        in_specs=[row, row, row, v128],
        out_specs=row,
        out_shape=jax.ShapeDtypeStruct((N, H), jnp.float32),
    )(s, o0, o1, pb)


def _rsqrt(x):
    i = lax.bitcast_convert_type(x, jnp.int32)
    i = jnp.int32(0x5F3759DF) - lax.shift_right_arithmetic(i, 1)
    y = lax.bitcast_convert_type(i, jnp.float32)
    for _ in range(3):
        y = y * (1.5 - 0.5 * x * y * y)
    return y


def _dist_body(src_hbm, dst_hbm, pos_hbm, gate_hbm, c_hbm, dist_hbm,
               pos_v, gate_v, c_v, sidx, didx, dbuf, sidx2, didx2, dbuf2,
               s_si, s_di, s_w):
    wid, _, _ = _wid()
    pltpu.sync_copy(pos_hbm, pos_v)
    pltpu.sync_copy(gate_hbm, gate_v)
    pltpu.sync_copy(c_hbm, c_v)
    cvec = c_v[...]
    GPT = N // NW
    NCH = GPT // 4
    ebase = wid * GPT * K
    gbase = wid * GPT

    def start_idx(c, k):
        off = ebase + c * 128
        pltpu.async_copy(src_hbm.at[pl.ds(off, 128)], sidx.at[k], s_si.at[k])
        pltpu.async_copy(dst_hbm.at[pl.ds(off, 128)], didx.at[k], s_di.at[k])

    def wait_idx(k):
        pltpu.make_async_copy(src_hbm.at[pl.ds(0, 128)], sidx.at[k], s_si.at[k]).wait()
        pltpu.make_async_copy(dst_hbm.at[pl.ds(0, 128)], didx.at[k], s_di.at[k]).wait()

    def wait_w(k):
        pltpu.make_async_copy(dbuf.at[k], dist_hbm.at[pl.ds(0, 128)], s_w.at[k]).wait()

    def edge_dist(idx_s, idx_d):
        si = idx_s * 3
        di = idx_d * 3
        dx = plsc.load_gather(pos_v, [si]) - plsc.load_gather(pos_v, [di])
        dy = plsc.load_gather(pos_v, [si + 1]) - plsc.load_gather(pos_v, [di + 1])
        dz = plsc.load_gather(pos_v, [si + 2]) - plsc.load_gather(pos_v, [di + 2])
        d2 = dx * dx + dy * dy + dz * dz + 1e-12
        return d2 * _rsqrt(d2)

    def process(c, k):
        wait_idx(k)

        @pl.when(c >= 2)
        def _ww():
            wait_w(k)
        gv = gate_v[pl.ds(gbase + c * 4, 16)]
        for q in range(4):
            dvals = []
            for rr in range(2):
                r = 2 * q + rr
                dvals.append(edge_dist(sidx[k, pl.ds(r * 16, 16)],
                                       didx[k, pl.ds(r * 16, 16)]))
            wsum = jnp.sum(cvec * dvals[0])
            filt = gv[q] * wsum
            dbuf[k, pl.ds((2 * q) * 16, 16)] = dvals[0] + filt
            dbuf[k, pl.ds((2 * q + 1) * 16, 16)] = dvals[1] + filt
        off = ebase + c * 128
        pltpu.async_copy(dbuf.at[k], dist_hbm.at[pl.ds(off, 128)], s_w.at[k])

        @pl.when(c + 2 < NCH)
        def _si():
            start_idx(c + 2, k)

    start_idx(0, 0)
    start_idx(1, 1)

    def pair(i, carry):
        process(2 * i, 0)
        process(2 * i + 1, 1)
        return carry

    lax.fori_loop(0, NCH // 2, pair, 0, unroll=False)
    wait_w(0)
    wait_w(1)

    @pl.when(wid < 16)
    def _extra():
        g = NW * GPT + wid
        off = g * K
        pltpu.sync_copy(src_hbm.at[pl.ds(off, K)], sidx2)
        pltpu.sync_copy(dst_hbm.at[pl.ds(off, K)], didx2)
        dvals = []
        for r in range(K // 16):
            dvals.append(edge_dist(sidx2[pl.ds(r * 16, 16)],
                                   didx2[pl.ds(r * 16, 16)]))
        wsum = jnp.sum(cvec * dvals[0])
        gv = gate_v[pl.ds(g, 16)]
        filt = gv[0] * wsum
        for r in range(K // 16):
            dbuf2[pl.ds(r * 16, 16)] = dvals[r] + filt
        pltpu.sync_copy(dbuf2, dist_hbm.at[pl.ds(off, K)])


def _sc_dist(src, dst, pos_flat, gate_flat, c_pad):
    return pl.kernel(
        _dist_body,
        out_type=jax.ShapeDtypeStruct((E,), jnp.float32),
        mesh=_MESH,
        compiler_params=_SC_PARAMS,
        scratch_types=[
            pltpu.VMEM((3 * N,), jnp.float32),
            pltpu.VMEM((N + 16,), jnp.float32),
            pltpu.VMEM((16,), jnp.float32),
            pltpu.VMEM((2, 128), jnp.int32),
            pltpu.VMEM((2, 128), jnp.int32),
            pltpu.VMEM((2, 128), jnp.float32),
            pltpu.VMEM((K,), jnp.int32),
            pltpu.VMEM((K,), jnp.int32),
            pltpu.VMEM((K,), jnp.float32),
            pltpu.SemaphoreType.DMA((2,)),
            pltpu.SemaphoreType.DMA((2,)),
            pltpu.SemaphoreType.DMA((2,)),
        ],
    )(src, dst, pos_flat, gate_flat, c_pad)


def _p1_body(src_hbm, dst_hbm, dist_hbm, xl_hbm, xr_hbm, wa_hbm,
             logits_hbm, tmax_hbm,
             wa_v, mx_v, sidx, didx, dist_v, lg_v, xlr, xrr,
             s_si, s_di, s_d, s_g1, s_g2, s_lg):
    wid, _, _ = _wid()
    base = wid * EPW
    pltpu.sync_copy(wa_hbm, wa_v)
    lane = lax.broadcasted_iota(jnp.int32, (16,), 0)

    def start_idx(c, k):
        off = base + c * CH
        pltpu.async_copy(src_hbm.at[pl.ds(off, CH)], sidx.at[k], s_si.at[k])
        pltpu.async_copy(dst_hbm.at[pl.ds(off, CH)], didx.at[k], s_di.at[k])

    def wait_idx(k):
        pltpu.make_async_copy(src_hbm.at[pl.ds(0, CH)], sidx.at[k], s_si.at[k]).wait()
        pltpu.make_async_copy(dst_hbm.at[pl.ds(0, CH)], didx.at[k], s_di.at[k]).wait()

    def start_dist(c, k):
        off = base + c * CH
        pltpu.async_copy(dist_hbm.at[pl.ds(off, CH)], dist_v.at[k], s_d.at[k])

    def wait_dist(k):
        pltpu.make_async_copy(dist_hbm.at[pl.ds(0, CH)], dist_v.at[k], s_d.at[k]).wait()

    def start_gather(k):
        pltpu.async_copy(xl_hbm.at[sidx.at[k]], xlr.at[k], s_g1.at[k])
        pltpu.async_copy(xr_hbm.at[didx.at[k]], xrr.at[k], s_g2.at[k])

    def wait_gather(k):
        pltpu.make_async_copy(xl_hbm.at[sidx.at[k]], xlr.at[k], s_g1.at[k]).wait()
        pltpu.make_async_copy(xr_hbm.at[didx.at[k]], xrr.at[k], s_g2.at[k]).wait()

    def wait_lg(k):
        pltpu.make_async_copy(lg_v.at[k], logits_hbm.at[pl.ds(0, CH)], s_lg.at[k]).wait()

    def compute(c, k, lmax):
        def grp(r, lmax):
            d_vec = dist_v[k, pl.ds(r * 16, 16)]
            lgvec = jnp.zeros((16,), jnp.float32)
            for l in range(16):
                e = r * 16 + l
                d = d_vec[l]
                acc = jnp.zeros((16,), jnp.float32)
                for j in range(H // 16):
                    z = xlr[k, e, pl.ds(j * 16, 16)] \
                        + xrr[k, e, pl.ds(j * 16, 16)] \
                        + wa_v[pl.ds(j * 16, 16)] * d
                    m = jnp.maximum(z, 0.2 * z)
                    acc = acc + wa_v[pl.ds(H + j * 16, 16)] * m
                lg = jnp.sum(acc)
                lgvec = jnp.where(lane == l, lg, lgvec)
                lmax = jnp.maximum(lmax, lg)
            lg_v[k, pl.ds(r * 16, 16)] = lgvec
            return lmax

        lmax = lax.fori_loop(0, CH // 16, grp, lmax, unroll=False)
        off = base + c * CH
        pltpu.async_copy(lg_v.at[k], logits_hbm.at[pl.ds(off, CH)], s_lg.at[k])
        return lmax

    def process(c, k, lmax, last=False):
        other = 1 - k
        if not last:
            wait_idx(other)
            start_gather(other)
        wait_gather(k)
        wait_dist(k)

        @pl.when(c + 2 < NCHUNK)
        def _starts():
            start_idx(c + 2, k)

        @pl.when(c >= 2)
        def _wlg():
            wait_lg(k)
        lmax = compute(c, k, lmax)

        @pl.when(c + 2 < NCHUNK)
        def _startd():
            start_dist(c + 2, k)
        return lmax

    start_idx(0, 0)
    start_idx(1, 1)
    start_dist(0, 0)
    start_dist(1, 1)
    wait_idx(0)
    start_gather(0)

    def pair(i, lmax):
        lmax = process(2 * i, 0, lmax)
        lmax = process(2 * i + 1, 1, lmax)
        return lmax

    lmax = lax.fori_loop(0, (NCHUNK - 1) // 2, pair, jnp.float32(-3e38),
                         unroll=False)
    lmax = process(NCHUNK - 1, 0, lmax, last=True)
    wait_lg(0)
    wait_lg(1)
    mx_v[...] = jnp.full((16,), lmax, jnp.float32)
    pltpu.sync_copy(mx_v, tmax_hbm.at[pl.ds(wid * 16, 16)])


def _sc_p1(src, dst, dist, xl, xr, wa):
    return pl.kernel(
        _p1_body,
        out_type=[
            jax.ShapeDtypeStruct((E,), jnp.float32),
            jax.ShapeDtypeStruct((NW * 16,), jnp.float32),
        ],
        mesh=_MESH,
        compiler_params=_SC_PARAMS,
        scratch_types=[
            pltpu.VMEM((2 * H,), jnp.float32),
            pltpu.VMEM((16,), jnp.float32),
            pltpu.VMEM((2, CH), jnp.int32),
            pltpu.VMEM((2, CH), jnp.int32),
            pltpu.VMEM((2, CH), jnp.float32),
            pltpu.VMEM((2, CH), jnp.float32),
            pltpu.VMEM((2, CH, H), jnp.float32),
            pltpu.VMEM((2, CH, H), jnp.float32),
            pltpu.SemaphoreType.DMA((2,)),
            pltpu.SemaphoreType.DMA((2,)),
            pltpu.SemaphoreType.DMA((2,)),
            pltpu.SemaphoreType.DMA((2,)),
            pltpu.SemaphoreType.DMA((2,)),
            pltpu.SemaphoreType.DMA((2,)),
        ],
    )(src, dst, dist, xl, xr, wa)


def _global_max(tm_v):
    m = tm_v[pl.ds(0, 16)]
    for i in range(1, NW):
        m = jnp.maximum(m, tm_v[pl.ds(i * 16, 16)])
    return jnp.max(m)


def _p2_body(dst_hbm, logits_hbm, tmax_hbm, esum_hbm,
             tm_v, didx, lg_v, ex_v, zb, s_di, s_lg, esp):
    wid, c, s = _wid()
    base = wid * EPW
    pltpu.sync_copy(tmax_hbm, tm_v)
    cmax = _global_max(tm_v)

    @pl.when(s == 0)
    def _zero():
        def zrow(i, carry):
            zb[pl.ds(i * 16, 16)] = jnp.zeros((16,), jnp.float32)
            return carry
        lax.fori_loop(0, N // 16, zrow, 0, unroll=False)
        pltpu.sync_copy(zb, esp)

    def start_loads(t, k):
        off = base + t * CH
        pltpu.async_copy(dst_hbm.at[pl.ds(off, CH)], didx.at[k], s_di.at[k])
        pltpu.async_copy(logits_hbm.at[pl.ds(off, CH)], lg_v.at[k], s_lg.at[k])

    def wait_loads(k):
        pltpu.make_async_copy(dst_hbm.at[pl.ds(0, CH)], didx.at[k], s_di.at[k]).wait()
        pltpu.make_async_copy(logits_hbm.at[pl.ds(0, CH)], lg_v.at[k], s_lg.at[k]).wait()

    start_loads(0, 0)
    start_loads(1, 1)
    plsc.subcore_barrier()

    def process(t, k):
        wait_loads(k)
        for r in range(CH // 16):
            ex_v[k, pl.ds(r * 16, 16)] = jnp.exp(lg_v[k, pl.ds(r * 16, 16)] - cmax)
        pltpu.sync_copy(ex_v.at[k], esp.at[didx.at[k]], add=True)

        @pl.when(t + 2 < NCHUNK)
        def _starts():
            start_loads(t + 2, k)

    def pair(i, carry):
        process(2 * i, 0)
        process(2 * i + 1, 1)
        return carry

    lax.fori_loop(0, (NCHUNK - 1) // 2, pair, 0, unroll=False)
    process(NCHUNK - 1, 0)
    plsc.subcore_barrier()

    @pl.when(s < 15)
    def _drain():
        pltpu.sync_copy(esp.at[pl.ds(s * 624, 624)], zb.at[pl.ds(0, 624)])
        pltpu.sync_copy(zb.at[pl.ds(0, 624)],
                        esum_hbm.at[pl.ds(c * N + s * 624, 624)])

    @pl.when(s == 15)
    def _drain_last():
        pltpu.sync_copy(esp.at[pl.ds(15 * 624, N - 15 * 624)],
                        zb.at[pl.ds(0, N - 15 * 624)])
        pltpu.sync_copy(zb.at[pl.ds(0, N - 15 * 624)],
                        esum_hbm.at[pl.ds(c * N + 15 * 624, N - 15 * 624)])


def _sc_p2(dst, logits, tmax):
    return pl.kernel(
        _p2_body,
        out_type=jax.ShapeDtypeStruct((NC * N,), jnp.float32),
        mesh=_MESH,
        compiler_params=_SC_PARAMS,
        scratch_types=[
            pltpu.VMEM((NW * 16,), jnp.float32),
            pltpu.VMEM((2, CH), jnp.int32),
            pltpu.VMEM((2, CH), jnp.float32),
            pltpu.VMEM((2, CH), jnp.float32),
            pltpu.VMEM((N,), jnp.float32),
            pltpu.SemaphoreType.DMA((2,)),
            pltpu.SemaphoreType.DMA((2,)),
            pltpu.VMEM_SHARED((N,), jnp.float32),
        ],
    )(dst, logits, tmax)


def _p3_body(src_hbm, dst_hbm, logits_hbm, tmax_hbm, esum_hbm, xl_hbm,
             out_hbm,
             tm_v, es_v, tmp_v, zb, sidx, didx, lg_v, xlr,
             s_si, s_di, s_lg, s_g1, osp):
    wid, c, s = _wid()
    base = wid * EPW
    pltpu.sync_copy(tmax_hbm, tm_v)
    cmax = _global_max(tm_v)

    pltpu.sync_copy(esum_hbm.at[pl.ds(0, N)], es_v)
    pltpu.sync_copy(esum_hbm.at[pl.ds(N, N)], tmp_v)

    def comb(i, carry):
        es_v[pl.ds(i * 16, 16)] = (es_v[pl.ds(i * 16, 16)]
                                   + tmp_v[pl.ds(i * 16, 16)] + 1e-16)
        return carry
    lax.fori_loop(0, N // 16, comb, 0, unroll=False)

    def zrow(i, carry):
        for j in range(H // 16):
            zb[i, pl.ds(j * 16, 16)] = jnp.zeros((16,), jnp.float32)
        return carry
    lax.fori_loop(0, 48, zrow, 0, unroll=False)
    for q in range(13):
        pltpu.sync_copy(zb, osp.at[pl.ds(s * 624 + q * 48, 48), :])

    @pl.when(s == 15)
    def _zero_tail():
        pltpu.sync_copy(zb.at[pl.ds(0, 32), :], osp.at[pl.ds(N - 32, 32), :])

    def start_loads(t, k):
        off = base + t * CH
        pltpu.async_copy(src_hbm.at[pl.ds(off, CH)], sidx.at[k], s_si.at[k])
        pltpu.async_copy(dst_hbm.at[pl.ds(off, CH)], didx.at[k], s_di.at[k])
        pltpu.async_copy(logits_hbm.at[pl.ds(off, CH)], lg_v.at[k], s_lg.at[k])

    def wait_loads(k):
        pltpu.make_async_copy(src_hbm.at[pl.ds(0, CH)], sidx.at[k], s_si.at[k]).wait()
        pltpu.make_async_copy(dst_hbm.at[pl.ds(0, CH)], didx.at[k], s_di.at[k]).wait()
        pltpu.make_async_copy(logits_hbm.at[pl.ds(0, CH)], lg_v.at[k], s_lg.at[k]).wait()

    def start_gather(k):
        pltpu.async_copy(xl_hbm.at[sidx.at[k]], xlr.at[k], s_g1.at[k])

    def wait_gather(k):
        pltpu.make_async_copy(xl_hbm.at[sidx.at[k]], xlr.at[k], s_g1.at[k]).wait()

    start_loads(0, 0)
    start_loads(1, 1)
    wait_loads(0)
    start_gather(0)
    plsc.subcore_barrier()

    def process(t, k, last=False):
        other = 1 - k
        if not last:
            wait_loads(other)
            start_gather(other)
        wait_gather(k)

        def grp(r, carry):
            ex = jnp.exp(lg_v[k, pl.ds(r * 16, 16)] - cmax)
            den = plsc.load_gather(es_v, [didx[k, pl.ds(r * 16, 16)]])
            a_vec = ex / den
            for l in range(16):
                e = r * 16 + l
                a = a_vec[l]
                for j in range(H // 16):
                    xlr[k, e, pl.ds(j * 16, 16)] = a * xlr[k, e, pl.ds(j * 16, 16)]
            return carry
        lax.fori_loop(0, CH // 16, grp, 0, unroll=False)
        pltpu.sync_copy(xlr.at[k], osp.at[didx.at[k]], add=True)

        @pl.when(t + 2 < NCHUNK)
        def _starts():
            start_loads(t + 2, k)

    def pair(i, carry):
        process(2 * i, 0)
        process(2 * i + 1, 1)
        return carry

    lax.fori_loop(0, (NCHUNK - 1) // 2, pair, 0, unroll=False)
    process(NCHUNK - 1, 0, last=True)
    plsc.subcore_barrier()

    for q in range(13):
        pltpu.sync_copy(osp.at[pl.ds(s * 624 + q * 48, 48), :], zb)
        pltpu.sync_copy(zb, out_hbm.at[pl.ds(c * N + s * 624 + q * 48, 48), :])

    @pl.when(s == 15)
    def _drain_tail():
        pltpu.sync_copy(osp.at[pl.ds(N - 32, 32), :], zb.at[pl.ds(0, 32), :])
        pltpu.sync_copy(zb.at[pl.ds(0, 32), :],
                        out_hbm.at[pl.ds(c * N + N - 32, 32), :])


def _sc_p3(src, dst, logits, tmax, esum, xl):
    return pl.kernel(
        _p3_body,
        out_type=jax.ShapeDtypeStruct((NC * N, H), jnp.float32),
        mesh=_MESH,
        compiler_params=_SC_PARAMS,
        scratch_types=[
            pltpu.VMEM((NW * 16,), jnp.float32),
            pltpu.VMEM((N,), jnp.float32),
            pltpu.VMEM((N,), jnp.float32),
            pltpu.VMEM((48, H), jnp.float32),
            pltpu.VMEM((2, CH), jnp.int32),
            pltpu.VMEM((2, CH), jnp.int32),
            pltpu.VMEM((2, CH), jnp.float32),
            pltpu.VMEM((2, CH, H), jnp.float32),
            pltpu.SemaphoreType.DMA((2,)),
            pltpu.SemaphoreType.DMA((2,)),
            pltpu.SemaphoreType.DMA((2,)),
            pltpu.SemaphoreType.DMA((2,)),
            pltpu.VMEM_SHARED((N, H), jnp.float32),
        ],
    )(src, dst, logits, tmax, esum, xl)


def kernel(x, edge_index, pos, s_emb_W, s_emb_b, frac_alpha, frac_gate_W,
           frac_gate_b, convs):
    src = edge_index[0]
    dst = edge_index[1]
    pos_flat = pos.reshape(-1)

    coeffs = [jnp.ones((), jnp.float32)]
    b = jnp.ones((), jnp.float32)
    for kk in range(1, WIN):
        b = b * (frac_alpha - (kk - 1)) / kk
        coeffs.append(((-1.0) ** kk) * b)
    c_pad = jnp.concatenate([jnp.stack(coeffs),
                             jnp.zeros((16 - WIN,), jnp.float32)])

    p0 = convs[0]
    s, gate, xl, xr = _tc_embed(
        x, s_emb_W.T, s_emb_b.reshape(1, H), frac_gate_W.reshape(1, H),
        frac_gate_b.reshape(1, 1), p0["Wl"].T, p0["bl"].reshape(1, H),
        p0["Wr"].T, p0["br"].reshape(1, H))

    gate_pad = jnp.concatenate([gate.reshape(-1), jnp.zeros((16,), jnp.float32)])
    dist = _sc_dist(src, dst, pos_flat, gate_pad, c_pad)

    outp = None
    for i, p in enumerate(convs):
        if i > 0:
            s, xl, xr = _tc_mid(s, outp[:N], outp[N:],
                                convs[i - 1]["bias"].reshape(1, H),
                                p["Wl"].T, p["bl"].reshape(1, H),
                                p["Wr"].T, p["br"].reshape(1, H))
        wa = jnp.concatenate([p["We"][:, 0], p["att"]])
        logits, tmax = _sc_p1(src, dst, dist, xl, xr, wa)
        esum = _sc_p2(dst, logits, tmax)
        outp = _sc_p3(src, dst, logits, tmax, esum, xl)

    s = _tc_post(s, outp[:N], outp[N:], convs[-1]["bias"].reshape(1, H))
    v = jnp.zeros((N, 16, 3), jnp.float32)
    return (s, v)


# speedup vs baseline: 16.1363x; 1.1005x over previous
import functools

import jax
import jax.numpy as jnp
from jax import lax
from jax.experimental import pallas as pl
from jax.experimental.pallas import tpu as pltpu
from jax.experimental.pallas import tpu_sc as plsc

N = 10000
E = 320000
H = 128
WIN = 5
K = E // N

NC = 2
NS = 16
NW = NC * NS
EPW = E // NW
CH = 80
NCHUNK = EPW // CH

_MESH = plsc.VectorSubcoreMesh(core_axis_name="c", subcore_axis_name="s")
_SC_PARAMS = pltpu.CompilerParams(needs_layout_passes=False)


def _wid():
    c = lax.axis_index("c")
    s = lax.axis_index("s")
    return s * NC + c, c, s


_BLK = 400
_GRID = N // _BLK


def _embed_body(x_ref, wt_ref, b_ref, gw_ref, gb_ref, wlt_ref, bl_ref,
                wrt_ref, br_ref, s_ref, gate_ref, xl_ref, xr_ref):
    s = jnp.dot(x_ref[...], wt_ref[...], preferred_element_type=jnp.float32)
    s = jnp.clip(s + b_ref[...], -10.0, 10.0)
    s_ref[...] = s
    gl = jnp.sum(s * gw_ref[...], axis=1, keepdims=True) + gb_ref[...]
    gate_ref[...] = 1.0 / (1.0 + jnp.exp(-gl))
    xl_ref[...] = jnp.dot(s, wlt_ref[...], preferred_element_type=jnp.float32) + bl_ref[...]
    xr_ref[...] = jnp.dot(s, wrt_ref[...], preferred_element_type=jnp.float32) + br_ref[...]


def _tc_embed(x, wt, b, gw, gb, wlt, bl, wrt, br):
    row = pl.BlockSpec((_BLK, H), lambda i: (i, 0))
    w128 = pl.BlockSpec((H, H), lambda i: (0, 0))
    v128 = pl.BlockSpec((1, H), lambda i: (0, 0))
    v1 = pl.BlockSpec((1, 1), lambda i: (0, 0))
    col = pl.BlockSpec((_BLK, 1), lambda i: (i, 0))
    return pl.pallas_call(
        _embed_body,
        grid=(_GRID,),
        in_specs=[row, w128, v128, v128, v1, w128, v128, w128, v128],
        out_specs=[row, col, row, row],
        out_shape=[
            jax.ShapeDtypeStruct((N, H), jnp.float32),
            jax.ShapeDtypeStruct((N, 1), jnp.float32),
            jax.ShapeDtypeStruct((N, H), jnp.float32),
            jax.ShapeDtypeStruct((N, H), jnp.float32),
        ],
    )(x, wt, b, gw, gb, wlt, bl, wrt, br)


def _mid_body(s_ref, o0_ref, o1_ref, pb_ref, wlt_ref, bl_ref, wrt_ref, br_ref,
              s_out_ref, xl_ref, xr_ref):
    s = s_ref[...] + o0_ref[...] + o1_ref[...] + pb_ref[...]
    s = jnp.maximum(jnp.clip(s, -20.0, 20.0), 0.0)
    s_out_ref[...] = s
    xl_ref[...] = jnp.dot(s, wlt_ref[...], preferred_element_type=jnp.float32) + bl_ref[...]
    xr_ref[...] = jnp.dot(s, wrt_ref[...], preferred_element_type=jnp.float32) + br_ref[...]


def _tc_mid(s, o0, o1, pb, wlt, bl, wrt, br):
    row = pl.BlockSpec((_BLK, H), lambda i: (i, 0))
    w128 = pl.BlockSpec((H, H), lambda i: (0, 0))
    v128 = pl.BlockSpec((1, H), lambda i: (0, 0))
    return pl.pallas_call(
        _mid_body,
        grid=(_GRID,),
        in_specs=[row, row, row, v128, w128, v128, w128, v128],
        out_specs=[row, row, row],
        out_shape=[
            jax.ShapeDtypeStruct((N, H), jnp.float32),
            jax.ShapeDtypeStruct((N, H), jnp.float32),
            jax.ShapeDtypeStruct((N, H), jnp.float32),
        ],
    )(s, o0, o1, pb, wlt, bl, wrt, br)


def _post_body(s_ref, o0_ref, o1_ref, pb_ref, s_out_ref):
    s = s_ref[...] + o0_ref[...] + o1_ref[...] + pb_ref[...]
    s_out_ref[...] = jnp.maximum(jnp.clip(s, -20.0, 20.0), 0.0)


def _tc_post(s, o0, o1, pb):
    row = pl.BlockSpec((_BLK, H), lambda i: (i, 0))
    v128 = pl.BlockSpec((1, H), lambda i: (0, 0))
    return pl.pallas_call(
        _post_body,
        grid=(_GRID,),
        in_specs=[row, row, row, v128],
        out_specs=row,
        out_shape=jax.ShapeDtypeStruct((N, H), jnp.float32),
    )(s, o0, o1, pb)


def _rsqrt(x):
    i = lax.bitcast_convert_type(x, jnp.int32)
    i = jnp.int32(0x5F3759DF) - lax.shift_right_arithmetic(i, 1)
    y = lax.bitcast_convert_type(i, jnp.float32)
    for _ in range(3):
        y = y * (1.5 - 0.5 * x * y * y)
    return y


def _dist_body(src_hbm, dst_hbm, pos_hbm, gate_hbm, c_hbm, dist_hbm,
               pos_v, gate_v, c_v, sidx, didx, dbuf, sidx2, didx2, dbuf2,
               s_si, s_di, s_w):
    wid, _, _ = _wid()
    pltpu.sync_copy(pos_hbm, pos_v)
    pltpu.sync_copy(gate_hbm, gate_v)
    pltpu.sync_copy(c_hbm, c_v)
    cvec = c_v[...]
    GPT = N // NW
    NCH = GPT // 4
    ebase = wid * GPT * K
    gbase = wid * GPT

    def start_idx(c, k):
        off = ebase + c * 128
        pltpu.async_copy(src_hbm.at[pl.ds(off, 128)], sidx.at[k], s_si.at[k])
        pltpu.async_copy(dst_hbm.at[pl.ds(off, 128)], didx.at[k], s_di.at[k])

    def wait_idx(k):
        pltpu.make_async_copy(src_hbm.at[pl.ds(0, 128)], sidx.at[k], s_si.at[k]).wait()
        pltpu.make_async_copy(dst_hbm.at[pl.ds(0, 128)], didx.at[k], s_di.at[k]).wait()

    def wait_w(k):
        pltpu.make_async_copy(dbuf.at[k], dist_hbm.at[pl.ds(0, 128)], s_w.at[k]).wait()

    def edge_dist(idx_s, idx_d):
        si = idx_s * 3
        di = idx_d * 3
        dx = plsc.load_gather(pos_v, [si]) - plsc.load_gather(pos_v, [di])
        dy = plsc.load_gather(pos_v, [si + 1]) - plsc.load_gather(pos_v, [di + 1])
        dz = plsc.load_gather(pos_v, [si + 2]) - plsc.load_gather(pos_v, [di + 2])
        d2 = dx * dx + dy * dy + dz * dz + 1e-12
        return d2 * _rsqrt(d2)

    def process(c, k):
        wait_idx(k)

        @pl.when(c >= 2)
        def _ww():
            wait_w(k)
        gv = gate_v[pl.ds(gbase + c * 4, 16)]
        for q in range(4):
            dvals = []
            for rr in range(2):
                r = 2 * q + rr
                dvals.append(edge_dist(sidx[k, pl.ds(r * 16, 16)],
                                       didx[k, pl.ds(r * 16, 16)]))
            wsum = jnp.sum(cvec * dvals[0])
            filt = gv[q] * wsum
            dbuf[k, pl.ds((2 * q) * 16, 16)] = dvals[0] + filt
            dbuf[k, pl.ds((2 * q + 1) * 16, 16)] = dvals[1] + filt
        off = ebase + c * 128
        pltpu.async_copy(dbuf.at[k], dist_hbm.at[pl.ds(off, 128)], s_w.at[k])

        @pl.when(c + 2 < NCH)
        def _si():
            start_idx(c + 2, k)

    start_idx(0, 0)
    start_idx(1, 1)

    def pair(i, carry):
        process(2 * i, 0)
        process(2 * i + 1, 1)
        return carry

    lax.fori_loop(0, NCH // 2, pair, 0, unroll=False)
    wait_w(0)
    wait_w(1)

    @pl.when(wid < 16)
    def _extra():
        g = NW * GPT + wid
        off = g * K
        pltpu.sync_copy(src_hbm.at[pl.ds(off, K)], sidx2)
        pltpu.sync_copy(dst_hbm.at[pl.ds(off, K)], didx2)
        dvals = []
        for r in range(K // 16):
            dvals.append(edge_dist(sidx2[pl.ds(r * 16, 16)],
                                   didx2[pl.ds(r * 16, 16)]))
        wsum = jnp.sum(cvec * dvals[0])
        gv = gate_v[pl.ds(g, 16)]
        filt = gv[0] * wsum
        for r in range(K // 16):
            dbuf2[pl.ds(r * 16, 16)] = dvals[r] + filt
        pltpu.sync_copy(dbuf2, dist_hbm.at[pl.ds(off, K)])


def _sc_dist(src, dst, pos_flat, gate_flat, c_pad):
    return pl.kernel(
        _dist_body,
        out_type=jax.ShapeDtypeStruct((E,), jnp.float32),
        mesh=_MESH,
        compiler_params=_SC_PARAMS,
        scratch_types=[
            pltpu.VMEM((3 * N,), jnp.float32),
            pltpu.VMEM((N + 16,), jnp.float32),
            pltpu.VMEM((16,), jnp.float32),
            pltpu.VMEM((2, 128), jnp.int32),
            pltpu.VMEM((2, 128), jnp.int32),
            pltpu.VMEM((2, 128), jnp.float32),
            pltpu.VMEM((K,), jnp.int32),
            pltpu.VMEM((K,), jnp.int32),
            pltpu.VMEM((K,), jnp.float32),
            pltpu.SemaphoreType.DMA((2,)),
            pltpu.SemaphoreType.DMA((2,)),
            pltpu.SemaphoreType.DMA((2,)),
        ],
    )(src, dst, pos_flat, gate_flat, c_pad)


def _p1_body(src_hbm, dst_hbm, dist_hbm, xl_hbm, xr_hbm, wa_hbm,
             logits_hbm, tmax_hbm,
             wa_v, mx_v, sidx, didx, dist_v, lg_v, xlr, xrr,
             s_si, s_di, s_d, s_g1, s_g2, s_lg):
    wid, _, _ = _wid()
    base = wid * EPW
    pltpu.sync_copy(wa_hbm, wa_v)
    lane = lax.broadcasted_iota(jnp.int32, (16,), 0)

    def start_idx(c, k):
        off = base + c * CH
        pltpu.async_copy(src_hbm.at[pl.ds(off, CH)], sidx.at[k], s_si.at[k])
        pltpu.async_copy(dst_hbm.at[pl.ds(off, CH)], didx.at[k], s_di.at[k])

    def wait_idx(k):
        pltpu.make_async_copy(src_hbm.at[pl.ds(0, CH)], sidx.at[k], s_si.at[k]).wait()
        pltpu.make_async_copy(dst_hbm.at[pl.ds(0, CH)], didx.at[k], s_di.at[k]).wait()

    def start_dist(c, k):
        off = base + c * CH
        pltpu.async_copy(dist_hbm.at[pl.ds(off, CH)], dist_v.at[k], s_d.at[k])

    def wait_dist(k):
        pltpu.make_async_copy(dist_hbm.at[pl.ds(0, CH)], dist_v.at[k], s_d.at[k]).wait()

    def start_gather(k):
        pltpu.async_copy(xl_hbm.at[sidx.at[k]], xlr.at[k], s_g1.at[k])
        pltpu.async_copy(xr_hbm.at[didx.at[k]], xrr.at[k], s_g2.at[k])

    def wait_gather(k):
        pltpu.make_async_copy(xl_hbm.at[sidx.at[k]], xlr.at[k], s_g1.at[k]).wait()
        pltpu.make_async_copy(xr_hbm.at[didx.at[k]], xrr.at[k], s_g2.at[k]).wait()

    def wait_lg(k):
        pltpu.make_async_copy(lg_v.at[k], logits_hbm.at[pl.ds(0, CH)], s_lg.at[k]).wait()

    def compute(c, k, lmax):
        def grp(r, lmax):
            d_vec = dist_v[k, pl.ds(r * 16, 16)]
            lgvec = jnp.zeros((16,), jnp.float32)
            for l in range(16):
                e = r * 16 + l
                d = d_vec[l]
                acc = jnp.zeros((16,), jnp.float32)
                for j in range(H // 16):
                    z = xlr[k, e, pl.ds(j * 16, 16)] \
                        + xrr[k, e, pl.ds(j * 16, 16)] \
                        + wa_v[pl.ds(j * 16, 16)] * d
                    m = jnp.maximum(z, 0.2 * z)
                    acc = acc + wa_v[pl.ds(H + j * 16, 16)] * m
                lg = jnp.sum(acc)
                lgvec = jnp.where(lane == l, lg, lgvec)
                lmax = jnp.maximum(lmax, lg)
            lg_v[k, pl.ds(r * 16, 16)] = lgvec
            return lmax

        lmax = lax.fori_loop(0, CH // 16, grp, lmax, unroll=False)
        off = base + c * CH
        pltpu.async_copy(lg_v.at[k], logits_hbm.at[pl.ds(off, CH)], s_lg.at[k])
        return lmax

    def process(c, k, lmax, last=False):
        other = 1 - k
        if not last:
            wait_idx(other)
            start_gather(other)
        wait_gather(k)
        wait_dist(k)

        @pl.when(c + 2 < NCHUNK)
        def _starts():
            start_idx(c + 2, k)

        @pl.when(c >= 2)
        def _wlg():
            wait_lg(k)
        lmax = compute(c, k, lmax)

        @pl.when(c + 2 < NCHUNK)
        def _startd():
            start_dist(c + 2, k)
        return lmax

    start_idx(0, 0)
    start_idx(1, 1)
    start_dist(0, 0)
    start_dist(1, 1)
    wait_idx(0)
    start_gather(0)

    def pair(i, lmax):
        lmax = process(2 * i, 0, lmax)
        lmax = process(2 * i + 1, 1, lmax)
        return lmax

    lmax = lax.fori_loop(0, (NCHUNK - 1) // 2, pair, jnp.float32(-3e38),
                         unroll=False)
    lmax = process(NCHUNK - 1, 0, lmax, last=True)
    wait_lg(0)
    wait_lg(1)
    mx_v[...] = jnp.full((16,), lmax, jnp.float32)
    pltpu.sync_copy(mx_v, tmax_hbm.at[pl.ds(wid * 16, 16)])


def _sc_p1(src, dst, dist, xl, xr, wa):
    return pl.kernel(
        _p1_body,
        out_type=[
            jax.ShapeDtypeStruct((E,), jnp.float32),
            jax.ShapeDtypeStruct((NW * 16,), jnp.float32),
        ],
        mesh=_MESH,
        compiler_params=_SC_PARAMS,
        scratch_types=[
            pltpu.VMEM((2 * H,), jnp.float32),
            pltpu.VMEM((16,), jnp.float32),
            pltpu.VMEM((2, CH), jnp.int32),
            pltpu.VMEM((2, CH), jnp.int32),
            pltpu.VMEM((2, CH), jnp.float32),
            pltpu.VMEM((2, CH), jnp.float32),
            pltpu.VMEM((2, CH, H), jnp.float32),
            pltpu.VMEM((2, CH, H), jnp.float32),
            pltpu.SemaphoreType.DMA((2,)),
            pltpu.SemaphoreType.DMA((2,)),
            pltpu.SemaphoreType.DMA((2,)),
            pltpu.SemaphoreType.DMA((2,)),
            pltpu.SemaphoreType.DMA((2,)),
            pltpu.SemaphoreType.DMA((2,)),
        ],
    )(src, dst, dist, xl, xr, wa)


def _global_max(tm_v):
    m = tm_v[pl.ds(0, 16)]
    for i in range(1, NW):
        m = jnp.maximum(m, tm_v[pl.ds(i * 16, 16)])
    return jnp.max(m)


def _p2_body(dst_hbm, logits_hbm, tmax_hbm, esum_hbm,
             tm_v, didx, didx_s, lg_v, ex_v, zb, s_di, s_lg, s_sc, esp):
    wid, c, s = _wid()
    base = wid * EPW
    pltpu.sync_copy(tmax_hbm, tm_v)
    cmax = _global_max(tm_v)

    @pl.when(s == 0)
    def _zero():
        def zrow(i, carry):
            zb[pl.ds(i * 16, 16)] = jnp.zeros((16,), jnp.float32)
            return carry
        lax.fori_loop(0, N // 16, zrow, 0, unroll=False)
        pltpu.sync_copy(zb, esp)

    def start_loads(t, k):
        off = base + t * CH
        pltpu.async_copy(dst_hbm.at[pl.ds(off, CH)], didx.at[k], s_di.at[k])
        pltpu.async_copy(logits_hbm.at[pl.ds(off, CH)], lg_v.at[k], s_lg.at[k])

    def wait_loads(k):
        pltpu.make_async_copy(dst_hbm.at[pl.ds(0, CH)], didx.at[k], s_di.at[k]).wait()
        pltpu.make_async_copy(logits_hbm.at[pl.ds(0, CH)], lg_v.at[k], s_lg.at[k]).wait()

    start_loads(0, 0)
    start_loads(1, 1)
    plsc.subcore_barrier()

    def wait_scatter(k):
        pltpu.make_async_copy(ex_v.at[k], esp.at[didx_s.at[k]], s_sc.at[k]).wait()

    def process(t, k):
        wait_loads(k)

        @pl.when(t >= 2)
        def _ws():
            wait_scatter(k)
        for r in range(CH // 16):
            ex_v[k, pl.ds(r * 16, 16)] = jnp.exp(lg_v[k, pl.ds(r * 16, 16)] - cmax)
            didx_s[k, pl.ds(r * 16, 16)] = didx[k, pl.ds(r * 16, 16)]
        pltpu.async_copy(ex_v.at[k], esp.at[didx_s.at[k]], s_sc.at[k], add=True)

        @pl.when(t + 2 < NCHUNK)
        def _starts():
            start_loads(t + 2, k)

    def pair(i, carry):
        process(2 * i, 0)
        process(2 * i + 1, 1)
        return carry

    lax.fori_loop(0, (NCHUNK - 1) // 2, pair, 0, unroll=False)
    process(NCHUNK - 1, 0)
    wait_scatter(0)
    wait_scatter(1)
    plsc.subcore_barrier()

    @pl.when(s < 15)
    def _drain():
        pltpu.sync_copy(esp.at[pl.ds(s * 624, 624)], zb.at[pl.ds(0, 624)])
        pltpu.sync_copy(zb.at[pl.ds(0, 624)],
                        esum_hbm.at[pl.ds(c * N + s * 624, 624)])

    @pl.when(s == 15)
    def _drain_last():
        pltpu.sync_copy(esp.at[pl.ds(15 * 624, N - 15 * 624)],
                        zb.at[pl.ds(0, N - 15 * 624)])
        pltpu.sync_copy(zb.at[pl.ds(0, N - 15 * 624)],
                        esum_hbm.at[pl.ds(c * N + 15 * 624, N - 15 * 624)])


def _sc_p2(dst, logits, tmax):
    return pl.kernel(
        _p2_body,
        out_type=jax.ShapeDtypeStruct((NC * N,), jnp.float32),
        mesh=_MESH,
        compiler_params=_SC_PARAMS,
        scratch_types=[
            pltpu.VMEM((NW * 16,), jnp.float32),
            pltpu.VMEM((2, CH), jnp.int32),
            pltpu.VMEM((2, CH), jnp.int32),
            pltpu.VMEM((2, CH), jnp.float32),
            pltpu.VMEM((2, CH), jnp.float32),
            pltpu.VMEM((N,), jnp.float32),
            pltpu.SemaphoreType.DMA((2,)),
            pltpu.SemaphoreType.DMA((2,)),
            pltpu.SemaphoreType.DMA((2,)),
            pltpu.VMEM_SHARED((N,), jnp.float32),
        ],
    )(dst, logits, tmax)


def _p3_body(src_hbm, dst_hbm, logits_hbm, tmax_hbm, esum_hbm, xl_hbm,
             out_hbm,
             tm_v, es_v, tmp_v, zb, sidx, didx, didx_s, lg_v, xlr,
             s_si, s_di, s_lg, s_g1, s_sc, osp):
    wid, c, s = _wid()
    base = wid * EPW
    pltpu.sync_copy(tmax_hbm, tm_v)
    cmax = _global_max(tm_v)

    pltpu.sync_copy(esum_hbm.at[pl.ds(0, N)], es_v)
    pltpu.sync_copy(esum_hbm.at[pl.ds(N, N)], tmp_v)

    def comb(i, carry):
        es_v[pl.ds(i * 16, 16)] = (es_v[pl.ds(i * 16, 16)]
                                   + tmp_v[pl.ds(i * 16, 16)] + 1e-16)
        return carry
    lax.fori_loop(0, N // 16, comb, 0, unroll=False)

    def zrow(i, carry):
        for j in range(H // 16):
            zb[i, pl.ds(j * 16, 16)] = jnp.zeros((16,), jnp.float32)
        return carry
    lax.fori_loop(0, 48, zrow, 0, unroll=False)
    for q in range(13):
        pltpu.sync_copy(zb, osp.at[pl.ds(s * 624 + q * 48, 48), :])

    @pl.when(s == 15)
    def _zero_tail():
        pltpu.sync_copy(zb.at[pl.ds(0, 32), :], osp.at[pl.ds(N - 32, 32), :])

    def start_loads(t, k):
        off = base + t * CH
        pltpu.async_copy(src_hbm.at[pl.ds(off, CH)], sidx.at[k], s_si.at[k])
        pltpu.async_copy(dst_hbm.at[pl.ds(off, CH)], didx.at[k], s_di.at[k])
        pltpu.async_copy(logits_hbm.at[pl.ds(off, CH)], lg_v.at[k], s_lg.at[k])

    def wait_loads(k):
        pltpu.make_async_copy(src_hbm.at[pl.ds(0, CH)], sidx.at[k], s_si.at[k]).wait()
        pltpu.make_async_copy(dst_hbm.at[pl.ds(0, CH)], didx.at[k], s_di.at[k]).wait()
        pltpu.make_async_copy(logits_hbm.at[pl.ds(0, CH)], lg_v.at[k], s_lg.at[k]).wait()

    def start_gather(k):
        pltpu.async_copy(xl_hbm.at[sidx.at[k]], xlr.at[k], s_g1.at[k])

    def wait_gather(k):
        pltpu.make_async_copy(xl_hbm.at[sidx.at[k]], xlr.at[k], s_g1.at[k]).wait()

    start_loads(0, 0)
    start_loads(1, 1)
    wait_loads(0)
    start_gather(0)
    plsc.subcore_barrier()

    def wait_scatter(k):
        pltpu.make_async_copy(xlr.at[k], osp.at[didx_s.at[k]], s_sc.at[k]).wait()

    def process(t, k, last=False):
        other = 1 - k
        if not last:
            wait_loads(other)

            @pl.when(t >= 1)
            def _ws():
                wait_scatter(other)
            start_gather(other)
        wait_gather(k)

        def grp(r, carry):
            ex = jnp.exp(lg_v[k, pl.ds(r * 16, 16)] - cmax)
            dvec = didx[k, pl.ds(r * 16, 16)]
            didx_s[k, pl.ds(r * 16, 16)] = dvec
            den = plsc.load_gather(es_v, [dvec])
            a_vec = ex / den
            for l in range(16):
                e = r * 16 + l
                a = a_vec[l]
                for j in range(H // 16):
                    xlr[k, e, pl.ds(j * 16, 16)] = a * xlr[k, e, pl.ds(j * 16, 16)]
            return carry
        lax.fori_loop(0, CH // 16, grp, 0, unroll=False)
        pltpu.async_copy(xlr.at[k], osp.at[didx_s.at[k]], s_sc.at[k], add=True)

        @pl.when(t + 2 < NCHUNK)
        def _starts():
            start_loads(t + 2, k)

    def pair(i, carry):
        process(2 * i, 0)
        process(2 * i + 1, 1)
        return carry

    lax.fori_loop(0, (NCHUNK - 1) // 2, pair, 0, unroll=False)
    process(NCHUNK - 1, 0, last=True)
    wait_scatter(0)
    wait_scatter(1)
    plsc.subcore_barrier()

    for q in range(13):
        pltpu.sync_copy(osp.at[pl.ds(s * 624 + q * 48, 48), :], zb)
        pltpu.sync_copy(zb, out_hbm.at[pl.ds(c * N + s * 624 + q * 48, 48), :])

    @pl.when(s == 15)
    def _drain_tail():
        pltpu.sync_copy(osp.at[pl.ds(N - 32, 32), :], zb.at[pl.ds(0, 32), :])
        pltpu.sync_copy(zb.at[pl.ds(0, 32), :],
                        out_hbm.at[pl.ds(c * N + N - 32, 32), :])


def _sc_p3(src, dst, logits, tmax, esum, xl):
    return pl.kernel(
        _p3_body,
        out_type=jax.ShapeDtypeStruct((NC * N, H), jnp.float32),
        mesh=_MESH,
        compiler_params=_SC_PARAMS,
        scratch_types=[
            pltpu.VMEM((NW * 16,), jnp.float32),
            pltpu.VMEM((N,), jnp.float32),
            pltpu.VMEM((N,), jnp.float32),
            pltpu.VMEM((48, H), jnp.float32),
            pltpu.VMEM((2, CH), jnp.int32),
            pltpu.VMEM((2, CH), jnp.int32),
            pltpu.VMEM((2, CH), jnp.int32),
            pltpu.VMEM((2, CH), jnp.float32),
            pltpu.VMEM((2, CH, H), jnp.float32),
            pltpu.SemaphoreType.DMA((2,)),
            pltpu.SemaphoreType.DMA((2,)),
            pltpu.SemaphoreType.DMA((2,)),
            pltpu.SemaphoreType.DMA((2,)),
            pltpu.SemaphoreType.DMA((2,)),
            pltpu.VMEM_SHARED((N, H), jnp.float32),
        ],
    )(src, dst, logits, tmax, esum, xl)


def kernel(x, edge_index, pos, s_emb_W, s_emb_b, frac_alpha, frac_gate_W,
           frac_gate_b, convs):
    src = edge_index[0]
    dst = edge_index[1]
    pos_flat = pos.reshape(-1)

    coeffs = [jnp.ones((), jnp.float32)]
    b = jnp.ones((), jnp.float32)
    for kk in range(1, WIN):
        b = b * (frac_alpha - (kk - 1)) / kk
        coeffs.append(((-1.0) ** kk) * b)
    c_pad = jnp.concatenate([jnp.stack(coeffs),
                             jnp.zeros((16 - WIN,), jnp.float32)])

    p0 = convs[0]
    s, gate, xl, xr = _tc_embed(
        x, s_emb_W.T, s_emb_b.reshape(1, H), frac_gate_W.reshape(1, H),
        frac_gate_b.reshape(1, 1), p0["Wl"].T, p0["bl"].reshape(1, H),
        p0["Wr"].T, p0["br"].reshape(1, H))

    gate_pad = jnp.concatenate([gate.reshape(-1), jnp.zeros((16,), jnp.float32)])
    dist = _sc_dist(src, dst, pos_flat, gate_pad, c_pad)

    outp = None
    for i, p in enumerate(convs):
        if i > 0:
            s, xl, xr = _tc_mid(s, outp[:N], outp[N:],
                                convs[i - 1]["bias"].reshape(1, H),
                                p["Wl"].T, p["bl"].reshape(1, H),
                                p["Wr"].T, p["br"].reshape(1, H))
        wa = jnp.concatenate([p["We"][:, 0], p["att"]])
        logits, tmax = _sc_p1(src, dst, dist, xl, xr, wa)
        esum = _sc_p2(dst, logits, tmax)
        outp = _sc_p3(src, dst, logits, tmax, esum, xl)

    s = _tc_post(s, outp[:N], outp[N:], convs[-1]["bias"].reshape(1, H))
    v = jnp.zeros((N, 16, 3), jnp.float32)
    return (s, v)
